# baseline (speedup 1.0000x reference)
import jax
import jax.numpy as jnp
from jax.experimental import pallas as pl
from jax.experimental.pallas import tpu as pltpu

N, C, H, O = 512, 64, 128, 64
NBITS = 7
T = 64
NT = N // T

_NAMES = ('op0', 'red0', 'exp1', 'op1', 'red1', 'exp2', 'op2')


def _body(gates, x0, x1, x2a, x2b,
          op0_W1, op0_b1, op0_W2, op0_b2,
          red0_W1, red0_b1, red0_W2, red0_b2,
          exp1_W1, exp1_b1, exp1_W2, exp1_b2,
          op1_W1, op1_b1, op1_W2, op1_b2,
          red1_W1, red1_b1, red1_W2, red1_b2,
          exp2_W1, exp2_b1, exp2_W2, exp2_b2,
          op2_W1, op2_b1, op2_W2, op2_b2,
          out0, out1, out2, mx, mn):
    i = pl.program_id(0)
    j = pl.program_id(1)
    f32 = jnp.float32

    def g(k):
        return gates[k:k + 1, :O]

    def mlp(x, W1, b1, W2, b2):
        h = jnp.maximum(jnp.dot(x, W1[...], preferred_element_type=f32) + b1[...], 0.0)
        return jnp.dot(h, W2[...], preferred_element_type=f32) + b2[...]

    xa = x2a[...]

    @pl.when(j == 0)
    def _():
        mx[...] = jnp.zeros((T, C), f32)
        mn[...] = jnp.ones((T, C), f32)

    ra = jax.lax.broadcasted_iota(jnp.int32, (T, T, 1), 0) + i * T
    cb = jax.lax.broadcasted_iota(jnp.int32, (T, T, 1), 1) + j * T
    diag = ra == cb
    mx[...] = jnp.maximum(mx[...], jnp.max(jnp.where(diag, 0.0, xa), axis=1))
    mn[...] = jnp.minimum(mn[...], jnp.min(jnp.where(diag, 1.0, xa), axis=1))

    xbT = jnp.swapaxes(x2b[...], 0, 1)
    xcat = jnp.concatenate([xa, xbT], axis=-1).reshape(T * T, 2 * C)
    h2 = jnp.maximum(jnp.dot(xcat, op2_W1[...], preferred_element_type=f32) + op2_b1[...], 0.0)
    s2 = (jnp.dot(h2, op2_W2[...], preferred_element_type=f32) + op2_b2[...]) * g(6)

    x1i = x1[pl.ds(i * T, T), :]
    x1j = x1[pl.ds(j * T, T), :]
    a_part = jnp.dot(x1i, exp2_W1[0:C, :], preferred_element_type=f32)
    b_part = jnp.dot(x1j, exp2_W1[C:2 * C, :], preferred_element_type=f32)
    he = jnp.maximum(a_part[:, None, :] + b_part[None, :, :] + exp2_b1[...][None], 0.0)
    s2 = s2 + (jnp.dot(he.reshape(T * T, H), exp2_W2[...], preferred_element_type=f32)
               + exp2_b2[...]) * g(5)
    out2[...] = (jax.nn.sigmoid(s2) * g(9)).reshape(T, T, O)

    @pl.when(j == NT - 1)
    def _():
        red = jnp.concatenate([mx[...], mn[...]], axis=-1)
        s1 = mlp(red, red1_W1, red1_b1, red1_W2, red1_b2) * g(4)
        s1 = s1 + mlp(x1i, op1_W1, op1_b1, op1_W2, op1_b2) * g(3)
        s1 = s1 + mlp(x0[...], exp1_W1, exp1_b1, exp1_W2, exp1_b2) * g(2)
        out1[...] = jax.nn.sigmoid(s1) * g(8)

    @pl.when((i == 0) & (j == 0))
    def _():
        x1f = x1[...]
        r1 = jnp.concatenate([jnp.max(x1f, axis=0, keepdims=True),
                              jnp.min(x1f, axis=0, keepdims=True)], axis=-1)
        s0 = mlp(x0[...], op0_W1, op0_b1, op0_W2, op0_b2) * g(0)
        s0 = s0 + mlp(r1, red0_W1, red0_b1, red0_W2, red0_b2) * g(1)
        out0[...] = jax.nn.sigmoid(s0) * g(7)


def kernel(x0, x1, x2, params, action):
    f32 = jnp.float32
    x1s = x1.reshape(N, C)
    x2s = x2.reshape(N, N, C)

    a = jnp.asarray(action, jnp.int32)
    bf = [((a >> (NBITS - 1 - k)) & 1).astype(f32) for k in range(NBITS)]
    act0 = (bf[0] + bf[1] > 0).astype(f32)
    act1 = (bf[2] + bf[3] + bf[4] > 0).astype(f32)
    act2 = (bf[5] + bf[6] > 0).astype(f32)
    gvec = jnp.stack(bf + [act0, act1, act2] + [jnp.zeros(())] * 6)
    gates = jnp.broadcast_to(gvec[:, None], (16, 128)).astype(f32)

    weights = []
    wspecs = []
    for name in _NAMES:
        for suff in ('_W1', '_b1', '_W2', '_b2'):
            w = params[name + suff]
            if w.ndim == 1:
                w = w.reshape(1, -1)
            weights.append(w)
            wspecs.append(pl.BlockSpec(w.shape, lambda i, j: (0, 0)))

    out0, out1, out2 = pl.pallas_call(
        _body,
        grid=(NT, NT),
        in_specs=[
            pl.BlockSpec((16, 128), lambda i, j: (0, 0)),
            pl.BlockSpec((1, C), lambda i, j: (0, 0)),
            pl.BlockSpec((N, C), lambda i, j: (0, 0)),
            pl.BlockSpec((T, T, C), lambda i, j: (i, j, 0)),
            pl.BlockSpec((T, T, C), lambda i, j: (j, i, 0)),
        ] + wspecs,
        out_specs=[
            pl.BlockSpec((1, O), lambda i, j: (0, 0)),
            pl.BlockSpec((T, O), lambda i, j: (i, 0)),
            pl.BlockSpec((T, T, O), lambda i, j: (i, j, 0)),
        ],
        out_shape=[
            jax.ShapeDtypeStruct((1, O), f32),
            jax.ShapeDtypeStruct((N, O), f32),
            jax.ShapeDtypeStruct((N, N, O), f32),
        ],
        scratch_shapes=[
            pltpu.VMEM((T, C), f32),
            pltpu.VMEM((T, C), f32),
        ],
        compiler_params=pltpu.CompilerParams(
            dimension_semantics=("arbitrary", "arbitrary"),
        ),
    )(gates, x0, x1s, x2s, x2s, *weights)

    return out0, out1.reshape(1, N, O), out2.reshape(1, N, N, O)
